# Initial kernel scaffold; baseline (speedup 1.0000x reference)
import jax
import jax.numpy as jnp
from jax.experimental import pallas as pl


def kernel(x, table):
    raise NotImplementedError("write your pallas kernel here")


# speedup vs baseline: 5.0350x; 5.0350x over previous
import functools

import jax
import jax.numpy as jnp
from jax import lax
from jax.experimental import pallas as pl
from jax.experimental.pallas import tpu as pltpu
from jax.experimental.pallas import tpu_sc as plsc

VOCAB = 1000000
EMBED_DIM = 32
BATCH = 16384
HIST = 200

NC, NS = 2, 16
NW = NC * NS
IDXW = 128
CHUNK = 1024
IPC = CHUNK // IDXW

N = BATCH * HIST
B_PER_W = N // NW
NCHUNK = B_PER_W // CHUNK
IDX_ROWS = N // IDXW


def _body(table_hbm, idx_hbm, out_hbm,
          idx_v0, idx_v1, rows_v0, rows_v1,
          sem_i0, sem_i1, sem_g0, sem_g1, sem_o0, sem_o1):
    wid = lax.axis_index("s") * NC + lax.axis_index("c")
    idx_vs = (idx_v0, idx_v1)
    rows_vs = (rows_v0, rows_v1)
    sem_is = (sem_i0, sem_i1)
    sem_gs = (sem_g0, sem_g1)
    sem_os = (sem_o0, sem_o1)

    irow0 = wid * (B_PER_W // IDXW)
    row0 = wid * B_PER_W

    def start_idx_load(g, b):
        pltpu.async_copy(
            idx_hbm.at[pl.ds(irow0 + g * IPC, IPC), :], idx_vs[b], sem_is[b])

    def wait_idx(b):
        pltpu.make_async_copy(
            idx_hbm.at[pl.ds(0, IPC), :], idx_vs[b], sem_is[b]).wait()

    def start_gathers(b):
        for j in range(IPC):
            pltpu.async_copy(
                table_hbm.at[idx_vs[b].at[j]],
                rows_vs[b].at[pl.ds(j * IDXW, IDXW), :],
                sem_gs[b])

    def wait_gathers(b):
        for j in range(IPC):
            pltpu.make_async_copy(
                table_hbm.at[idx_vs[b].at[j]],
                rows_vs[b].at[pl.ds(j * IDXW, IDXW), :],
                sem_gs[b]).wait()

    def start_store(g, b):
        pltpu.async_copy(
            rows_vs[b], out_hbm.at[pl.ds(row0 + g * CHUNK, CHUNK), :],
            sem_os[b])

    def wait_store(b):
        pltpu.make_async_copy(
            rows_vs[b], out_hbm.at[pl.ds(row0, CHUNK), :], sem_os[b]).wait()

    def chunk_step(g, b):
        wait_idx(b)

        @pl.when(g >= 2)
        def _():
            wait_store(b)

        start_gathers(b)
        wait_gathers(b)
        start_store(g, b)

        @pl.when(g + 2 < NCHUNK)
        def _():
            start_idx_load(g + 2, b)

    start_idx_load(0, 0)
    start_idx_load(1, 1)

    def loop_body(t, carry):
        chunk_step(2 * t, 0)
        chunk_step(2 * t + 1, 1)
        return carry

    lax.fori_loop(0, NCHUNK // 2, loop_body, 0)

    wait_store(0)
    wait_store(1)


@functools.partial(jax.jit, donate_argnums=())
def _lookup(idx2d, table):
    mesh = plsc.VectorSubcoreMesh(
        core_axis_name="c", subcore_axis_name="s",
        num_cores=NC, num_subcores=NS)
    f = pl.kernel(
        _body,
        out_type=jax.ShapeDtypeStruct((N, EMBED_DIM), jnp.float32),
        mesh=mesh,
        compiler_params=pltpu.CompilerParams(use_tc_tiling_on_sc=False),
        scratch_types=[
            pltpu.VMEM((IPC, IDXW), jnp.int32),
            pltpu.VMEM((IPC, IDXW), jnp.int32),
            pltpu.VMEM((CHUNK, EMBED_DIM), jnp.float32),
            pltpu.VMEM((CHUNK, EMBED_DIM), jnp.float32),
            pltpu.SemaphoreType.DMA,
            pltpu.SemaphoreType.DMA,
            pltpu.SemaphoreType.DMA,
            pltpu.SemaphoreType.DMA,
            pltpu.SemaphoreType.DMA,
            pltpu.SemaphoreType.DMA,
        ],
    )
    return f(table, idx2d)


def kernel(x, table):
    idx2d = x.astype(jnp.int32).reshape(IDX_ROWS, IDXW)
    out = _lookup(idx2d, table)
    return out.reshape(BATCH, HIST, EMBED_DIM)


# speedup vs baseline: 5.0489x; 1.0028x over previous
import functools

import jax
import jax.numpy as jnp
from jax import lax
from jax.experimental import pallas as pl
from jax.experimental.pallas import tpu as pltpu
from jax.experimental.pallas import tpu_sc as plsc

VOCAB = 1000000
EMBED_DIM = 32
BATCH = 16384
HIST = 200

NC, NS = 2, 16
NW = NC * NS
IDXW = 128
CHUNK = 1024
IPC = CHUNK // IDXW

N = BATCH * HIST
B_PER_W = N // NW
NCHUNK = B_PER_W // CHUNK
IDX_ROWS = N // IDXW


def _body(table_hbm, idx_hbm, out_hbm,
          idx_v0, idx_v1, rows_v0, rows_v1,
          sem_i0, sem_i1, sem_g0, sem_g1, sem_o0, sem_o1):
    wid = lax.axis_index("s") * NC + lax.axis_index("c")
    idx_vs = (idx_v0, idx_v1)
    rows_vs = (rows_v0, rows_v1)
    sem_is = (sem_i0, sem_i1)
    sem_gs = (sem_g0, sem_g1)
    sem_os = (sem_o0, sem_o1)

    irow0 = wid * (B_PER_W // IDXW)
    row0 = wid * B_PER_W

    def start_idx_load(g, b):
        pltpu.async_copy(
            idx_hbm.at[pl.ds(irow0 + g * IPC, IPC), :], idx_vs[b], sem_is[b])

    def wait_idx(b):
        pltpu.make_async_copy(
            idx_hbm.at[pl.ds(0, IPC), :], idx_vs[b], sem_is[b]).wait()

    def start_gathers(b):
        for j in range(IPC):
            pltpu.async_copy(
                table_hbm.at[idx_vs[b].at[j]],
                rows_vs[b].at[pl.ds(j * IDXW, IDXW), :],
                sem_gs[b])

    def wait_gathers(b):
        for j in range(IPC):
            pltpu.make_async_copy(
                table_hbm.at[idx_vs[b].at[j]],
                rows_vs[b].at[pl.ds(j * IDXW, IDXW), :],
                sem_gs[b]).wait()

    def start_store(g, b):
        pltpu.async_copy(
            rows_vs[b], out_hbm.at[pl.ds(row0 + g * CHUNK, CHUNK), :],
            sem_os[b])

    def wait_store(b):
        pltpu.make_async_copy(
            rows_vs[b], out_hbm.at[pl.ds(row0, CHUNK), :], sem_os[b]).wait()

    def half_step(g, b):
        wait_idx(b)

        @pl.when(g >= 2)
        def _():
            wait_store(b)

        start_gathers(b)

        @pl.when(g >= 1)
        def _():
            wait_gathers(1 - b)
            start_store(g - 1, 1 - b)

            @pl.when(g + 1 < NCHUNK)
            def _():
                start_idx_load(g + 1, 1 - b)

    start_idx_load(0, 0)
    start_idx_load(1, 1)

    def loop_body(t, carry):
        half_step(2 * t, 0)
        half_step(2 * t + 1, 1)
        return carry

    lax.fori_loop(0, NCHUNK // 2, loop_body, 0)

    wait_gathers(1)
    start_store(NCHUNK - 1, 1)
    wait_store(0)
    wait_store(1)


@functools.partial(jax.jit, donate_argnums=())
def _lookup(idx2d, table):
    mesh = plsc.VectorSubcoreMesh(
        core_axis_name="c", subcore_axis_name="s",
        num_cores=NC, num_subcores=NS)
    f = pl.kernel(
        _body,
        out_type=jax.ShapeDtypeStruct((N, EMBED_DIM), jnp.float32),
        mesh=mesh,
        compiler_params=pltpu.CompilerParams(use_tc_tiling_on_sc=False),
        scratch_types=[
            pltpu.VMEM((IPC, IDXW), jnp.int32),
            pltpu.VMEM((IPC, IDXW), jnp.int32),
            pltpu.VMEM((CHUNK, EMBED_DIM), jnp.float32),
            pltpu.VMEM((CHUNK, EMBED_DIM), jnp.float32),
            pltpu.SemaphoreType.DMA,
            pltpu.SemaphoreType.DMA,
            pltpu.SemaphoreType.DMA,
            pltpu.SemaphoreType.DMA,
            pltpu.SemaphoreType.DMA,
            pltpu.SemaphoreType.DMA,
        ],
    )
    return f(table, idx2d)


def kernel(x, table):
    idx2d = x.astype(jnp.int32).reshape(IDX_ROWS, IDXW)
    out = _lookup(idx2d, table)
    return out.reshape(BATCH, HIST, EMBED_DIM)
